# baseline (speedup 1.0000x reference)
import functools

import jax
import jax.numpy as jnp
from jax import lax
from jax.experimental import pallas as pl
from jax.experimental.pallas import tpu as pltpu
from jax.experimental.pallas import tpu_sc as plsc

_F32 = jnp.float32
_BF16 = jnp.bfloat16
_SQRT1_2 = 0.7071067811865476

_BN_BLOCK = 128
_GATHER_CHUNK = 128


def _gelu(x):
    return 0.5 * x * (1.0 + lax.erf(x * _SQRT1_2))


def _ln(x, g, b):
    mu = jnp.mean(x, axis=-1, keepdims=True)
    d = x - mu
    var = jnp.mean(d * d, axis=-1, keepdims=True)
    return g * (d / jnp.sqrt(var + 1e-5)) + b


def _mm(x, w):
    return jnp.dot(x.astype(_BF16), w, preferred_element_type=_F32)


def _unpack2(x):
    lo = lax.bitcast_convert_type(x << 16, _F32).astype(_BF16)
    hi = lax.bitcast_convert_type(x & jnp.int32(-65536), _F32).astype(_BF16)
    return lo, hi


def _sc_gather(table, idx):
    M = idx.shape[0]
    D = table.shape[1]
    n_workers = 32
    per_w = M // n_workers
    n_chunks = per_w // _GATHER_CHUNK
    mesh = plsc.VectorSubcoreMesh(core_axis_name="c", subcore_axis_name="s")

    CH = _GATHER_CHUNK

    @functools.partial(
        pl.kernel,
        mesh=mesh,
        out_type=jax.ShapeDtypeStruct((M, D), table.dtype),
        scratch_types=[
            pltpu.VMEM((per_w,), jnp.int32),
            pltpu.VMEM((CH, D), table.dtype),
            pltpu.VMEM((CH, D), table.dtype),
            pltpu.SemaphoreType.DMA,
            pltpu.SemaphoreType.DMA,
            pltpu.SemaphoreType.DMA,
            pltpu.SemaphoreType.DMA,
        ],
    )
    def k(table_hbm, idx_hbm, out_hbm, idx_v, rows0, rows1, g0, g1, w0, w1):
        wid = lax.axis_index("s") * 2 + lax.axis_index("c")
        base = wid * per_w
        rows = (rows0, rows1)
        gsem = (g0, g1)
        wsem = (w0, w1)
        pltpu.sync_copy(idx_hbm.at[pl.ds(base, per_w)], idx_v)
        pend_g = [None, None]
        pend_w = [None, None]
        for i in range(n_chunks):
            b = i % 2
            if pend_w[b] is not None:
                pend_w[b].wait()
            pend_g[b] = pltpu.async_copy(
                table_hbm.at[idx_v.at[pl.ds(i * CH, CH)]], rows[b], gsem[b])
            if i >= 1:
                pb = 1 - b
                pend_g[pb].wait()
                pend_w[pb] = pltpu.async_copy(
                    rows[pb],
                    out_hbm.at[pl.ds(base + (i - 1) * CH, CH)], wsem[pb])
        bl = (n_chunks - 1) % 2
        pend_g[bl].wait()
        pltpu.sync_copy(rows[bl],
                        out_hbm.at[pl.ds(base + (n_chunks - 1) * CH, CH)])
        if pend_w[1 - bl] is not None:
            pend_w[1 - bl].wait()

    return k(table, idx)


def _node_spec(bn, h):
    return pl.BlockSpec((bn, h), lambda i: (i, 0))


def _full_spec(shape):
    return pl.BlockSpec(shape, lambda i: tuple(0 for _ in shape))


def _embed_body(x_ref, wt_ref, b_ref, of_ref, ob_ref, *, K):
    he = _mm(x_ref[...], wt_ref[...]) + b_ref[...]
    of_ref[...] = he
    ob_ref[...] = he.astype(_BF16)


def _edge_mlp(x1, w2, b2, w3, b3):
    e1 = _gelu(x1.astype(_BF16))
    e2 = _gelu((jnp.dot(e1, w2[...], preferred_element_type=_F32)
                + b2[...]).astype(_BF16))
    return jnp.dot(e2, w3[...], preferred_element_type=_F32) + b3[...]


def _node_update(hv, dh, mv, g1, bb1, win, bin_, wo4, bo, g2, bb2):
    u = _ln(hv + dh, g1[...], bb1[...])
    f1 = _gelu(_mm(u, win[...]) + bin_[...])
    v = _ln(u + _mm(f1, wo4[...]) + bo[...], g2[...], bb2[...])
    return v * mv[...]


def _enc_a0_body(heb_ref, ma_ref, mv_ref,
                 w1b, b1, w2, b2, w3, b3,
                 g1, bb1, win, bin_, wo4, bo, g2, bb2, out_ref, *, K):
    bnk, H = heb_ref.shape
    bn = bnk // K
    t = _mm(heb_ref[...], w1b[...]) + b1[...]
    m = _edge_mlp(t, w2, b2, w3, b3)
    m = m.reshape(bn, K, H) * ma_ref[...][:, :, None]
    dh = jnp.sum(m, axis=1) * (1.0 / 30.0)
    u = _ln(dh, g1[...], bb1[...])
    f1 = _gelu(_mm(u, win[...]) + bin_[...])
    v = _ln(u + _mm(f1, wo4[...]) + bo[...], g2[...], bb2[...])
    out_ref[...] = v * mv_ref[...]


def _enc_a_body(hv_ref, heb_ref, hvg_ref, ma_ref, mv_ref,
                w1a, w1b, w1c, b1, w2, b2, w3, b3,
                g1, bb1, win, bin_, wo4, bo, g2, bb2, out_ref, *, K):
    bn, H = hv_ref.shape
    hv = hv_ref[...]
    t = _mm(heb_ref[...], w1b[...]) + _mm(hvg_ref[...], w1c[...])
    hv1 = _mm(hv, w1a[...])
    x1 = t.reshape(bn, K, H) + hv1[:, None, :] + b1[...][None]
    m = _edge_mlp(x1.reshape(bn * K, H), w2, b2, w3, b3)
    m = m.reshape(bn, K, H) * ma_ref[...][:, :, None]
    dh = jnp.sum(m, axis=1) * (1.0 / 30.0)
    out_ref[...] = _node_update(hv, dh, mv_ref, g1, bb1, win, bin_, wo4, bo,
                                g2, bb2)


def _enc_b_body(hv_ref, hef_ref, hvg_ref,
                w1a, w1b, w1c, b1, w2, b2, w3, b3, g3, bb3,
                of_ref, ob_ref, *, K):
    bn, H = hv_ref.shape
    hef = hef_ref[...]
    t = _mm(hef, w1b[...]) + _mm(hvg_ref[...], w1c[...])
    hv1 = _mm(hv_ref[...], w1a[...])
    x1 = t.reshape(bn, K, H) + hv1[:, None, :] + b1[...][None]
    m = _edge_mlp(x1.reshape(bn * K, H), w2, b2, w3, b3)
    he = _ln(hef + m, g3[...], bb3[...])
    of_ref[...] = he
    ob_ref[...] = he.astype(_BF16)


def _dec_body(hv_ref, heb_ref, hsg_ref, hveg_ref, hvg_ref, m2_ref, mv_ref,
              w1a, w1b, w1c, w1d, b1, w2, b2, w3, b3,
              g1, bb1, win, bin_, wo4, bo, g2, bb2, out_ref, *, K):
    bn, H = hv_ref.shape
    hv = hv_ref[...]
    mvb = mv_ref[...].astype(_BF16)[:, :, None]
    m2b = m2_ref[...].astype(_BF16)[:, :, None]
    m2e = m2b * mvb
    fwe = mvb - m2e
    heb = heb_ref[...].reshape(bn, K, H)
    xb = (heb * mvb).reshape(bn * K, H)
    hs_lo, hs_hi = _unpack2(hsg_ref[...])
    d = H // 2
    hveg = hveg_ref[...].astype(_BF16).reshape(bn, K, H)
    hvg = hvg_ref[...].astype(_BF16).reshape(bn, K, H)

    def mask3(x, m):
        return (x.reshape(bn, K, x.shape[-1]) * m).reshape(bn * K, -1)

    w1cw = w1c[...]
    xd = (hvg * m2e + hveg * fwe).reshape(bn * K, H)
    t = (jnp.dot(xb, w1b[...], preferred_element_type=_F32)
         + jnp.dot(mask3(hs_lo, m2e), w1cw[:d], preferred_element_type=_F32)
         + jnp.dot(mask3(hs_hi, m2e), w1cw[d:], preferred_element_type=_F32)
         + jnp.dot(xd, w1d[...], preferred_element_type=_F32))
    hv1 = _mm(hv, w1a[...])
    x1 = t.reshape(bn, K, H) + hv1[:, None, :] + b1[...][None]
    m = _edge_mlp(x1.reshape(bn * K, H), w2, b2, w3, b3)
    dh = jnp.sum(m.reshape(bn, K, H), axis=1) * (1.0 / 30.0)
    out_ref[...] = _node_update(hv, dh, mv_ref, g1, bb1, win, bin_, wo4, bo,
                                g2, bb2)


def _probs_body(hv_ref, wo_ref, bo_ref, out_ref):
    logits = _mm(hv_ref[...], wo_ref[...]) + bo_ref[...]
    mx = jnp.max(logits, axis=-1, keepdims=True)
    e = jnp.exp(logits - mx)
    out_ref[...] = e / jnp.sum(e, axis=-1, keepdims=True)


def _wt(p):
    return p["W"].T.astype(_BF16)


def _b2d(p):
    return p["b"][None, :].astype(_F32)


def _n2d(p):
    return p["g"][None, :].astype(_F32), p["b"][None, :].astype(_F32)


def kernel(X, S, mask, E_idx, params):
    B, N, K, EF = X.shape
    H = params["W_e"]["W"].shape[0]
    V = params["W_out"]["W"].shape[0]
    BN = B * N
    M = BN * K
    bn = _BN_BLOCK
    grid = (BN // bn,)
    bnk = bn * K

    offs = (jnp.arange(B, dtype=jnp.int32) * N)[:, None, None]
    flat_idx = (E_idx + offs).reshape(-1)

    mv = mask.reshape(BN, 1)

    rand = jnp.abs(jax.random.normal(jax.random.key(42), (B, N), dtype=_F32))
    decoding_order = jnp.argsort((mask + 0.0001) * rand, axis=-1)
    iota_n = jnp.broadcast_to(jnp.arange(N, dtype=jnp.int32), (B, N))
    pos = (jnp.zeros((B, N), jnp.int32)
           .at[jnp.arange(B)[:, None], decoding_order].set(iota_n)
           .astype(_F32).reshape(BN))

    hs = jnp.take(params["W_s"], S.reshape(-1), axis=0).astype(_BF16)
    d2 = H // 2
    hs_lo = lax.bitcast_convert_type(hs[:, :d2], jnp.uint16).astype(jnp.uint32)
    hs_hi = lax.bitcast_convert_type(hs[:, d2:], jnp.uint16).astype(jnp.uint32)
    hs_pk = lax.bitcast_convert_type(hs_lo | (hs_hi << 16), _F32)
    packed = jnp.zeros((BN, 128), _F32)
    packed = (packed.at[:, :d2].set(hs_pk)
              .at[:, d2].set(mask.reshape(BN)).at[:, d2 + 1].set(pos))
    pg = _sc_gather(packed, flat_idx)
    mask_attend = mv * pg[:, d2].reshape(BN, K)
    m2 = (pos[:, None] > pg[:, d2 + 1].reshape(BN, K)).astype(_F32)
    hsg = lax.bitcast_convert_type(pg[:, :d2], jnp.int32)

    edge_spec = _node_spec(bnk, H)
    hspec = _node_spec(bn, H)
    w_hh = _full_spec((H, H))
    b_h = _full_spec((1, H))

    def enc_a0(heb, p):
        w1t = p["W1"]["W"].T.astype(_BF16)
        g1, bb1 = _n2d(p["norm1"])
        g2, bb2 = _n2d(p["norm2"])
        return pl.pallas_call(
            functools.partial(_enc_a0_body, K=K),
            grid=grid,
            in_specs=[edge_spec, _node_spec(bn, K), _node_spec(bn, 1),
                      w_hh, b_h, w_hh, b_h, w_hh, b_h,
                      b_h, b_h, _full_spec((H, 4 * H)), _full_spec((1, 4 * H)),
                      _full_spec((4 * H, H)), b_h, b_h, b_h],
            out_specs=hspec,
            out_shape=jax.ShapeDtypeStruct((BN, H), _F32),
        )(heb, mask_attend, mv,
          w1t[H:2 * H], _b2d(p["W1"]),
          _wt(p["W2"]), _b2d(p["W2"]), _wt(p["W3"]), _b2d(p["W3"]),
          g1, bb1, _wt(p["dense"]["W_in"]), _b2d(p["dense"]["W_in"]),
          _wt(p["dense"]["W_out"]), _b2d(p["dense"]["W_out"]), g2, bb2)

    def enc_a(hv, heb, hvg, p):
        w1t = p["W1"]["W"].T.astype(_BF16)
        g1, bb1 = _n2d(p["norm1"])
        g2, bb2 = _n2d(p["norm2"])
        return pl.pallas_call(
            functools.partial(_enc_a_body, K=K),
            grid=grid,
            in_specs=[hspec, edge_spec, edge_spec,
                      _node_spec(bn, K), _node_spec(bn, 1),
                      w_hh, w_hh, w_hh, b_h, w_hh, b_h, w_hh, b_h,
                      b_h, b_h, _full_spec((H, 4 * H)), _full_spec((1, 4 * H)),
                      _full_spec((4 * H, H)), b_h, b_h, b_h],
            out_specs=hspec,
            out_shape=jax.ShapeDtypeStruct((BN, H), _F32),
        )(hv, heb, hvg, mask_attend, mv,
          w1t[:H], w1t[H:2 * H], w1t[2 * H:], _b2d(p["W1"]),
          _wt(p["W2"]), _b2d(p["W2"]), _wt(p["W3"]), _b2d(p["W3"]),
          g1, bb1, _wt(p["dense"]["W_in"]), _b2d(p["dense"]["W_in"]),
          _wt(p["dense"]["W_out"]), _b2d(p["dense"]["W_out"]), g2, bb2)

    def enc_b(hv, hef, hvg, p):
        w1t = p["W11"]["W"].T.astype(_BF16)
        g3, bb3 = _n2d(p["norm3"])
        return pl.pallas_call(
            functools.partial(_enc_b_body, K=K),
            grid=grid,
            in_specs=[hspec, edge_spec, edge_spec,
                      w_hh, w_hh, w_hh, b_h, w_hh, b_h, w_hh, b_h, b_h, b_h],
            out_specs=[edge_spec, edge_spec],
            out_shape=[jax.ShapeDtypeStruct((M, H), _F32),
                       jax.ShapeDtypeStruct((M, H), _BF16)],
        )(hv, hef, hvg,
          w1t[:H], w1t[H:2 * H], w1t[2 * H:], _b2d(p["W11"]),
          _wt(p["W12"]), _b2d(p["W12"]), _wt(p["W13"]), _b2d(p["W13"]),
          g3, bb3)

    def dec(hv, heb, hsg, hveg, hvg, p):
        w1t = p["W1"]["W"].T.astype(_BF16)
        g1, bb1 = _n2d(p["norm1"])
        g2, bb2 = _n2d(p["norm2"])
        return pl.pallas_call(
            functools.partial(_dec_body, K=K),
            grid=grid,
            in_specs=[hspec, edge_spec, _node_spec(bnk, H // 2), edge_spec,
                      edge_spec, _node_spec(bn, K), _node_spec(bn, 1),
                      w_hh, w_hh, w_hh, w_hh, b_h, w_hh, b_h, w_hh, b_h,
                      b_h, b_h, _full_spec((H, 4 * H)), _full_spec((1, 4 * H)),
                      _full_spec((4 * H, H)), b_h, b_h, b_h],
            out_specs=hspec,
            out_shape=jax.ShapeDtypeStruct((BN, H), _F32),
        )(hv, heb, hsg, hveg, hvg, m2, mv,
          w1t[:H], w1t[H:2 * H], w1t[2 * H:3 * H], w1t[3 * H:], _b2d(p["W1"]),
          _wt(p["W2"]), _b2d(p["W2"]), _wt(p["W3"]), _b2d(p["W3"]),
          g1, bb1, _wt(p["dense"]["W_in"]), _b2d(p["dense"]["W_in"]),
          _wt(p["dense"]["W_out"]), _b2d(p["dense"]["W_out"]), g2, bb2)

    X2 = X.reshape(M, EF)
    wet = params["W_e"]["W"].T.astype(_BF16)
    hef, heb = pl.pallas_call(
        functools.partial(_embed_body, K=K),
        grid=grid,
        in_specs=[_node_spec(bnk, EF), _full_spec((EF, H)), b_h],
        out_specs=[edge_spec, edge_spec],
        out_shape=[jax.ShapeDtypeStruct((M, H), _F32),
                   jax.ShapeDtypeStruct((M, H), _BF16)],
    )(X2, wet, _b2d(params["W_e"]))

    hv = hvg = None
    for i, p in enumerate(params["enc"]):
        hv = enc_a0(heb, p) if i == 0 else enc_a(hv, heb, hvg, p)
        hvg = _sc_gather(hv, flat_idx)
        hef, heb = enc_b(hv, hef, hvg, p)

    hveg = hvg
    for i, p in enumerate(params["dec"]):
        hvg_d = hveg if i == 0 else _sc_gather(hv, flat_idx)
        hv = dec(hv, heb, hsg, hveg, hvg_d, p)

    wo = jnp.zeros((H, 128), _F32).at[:, :V].set(
        params["W_out"]["W"].T).astype(_BF16)
    bo = jnp.full((1, 128), -1e9, _F32).at[0, :V].set(params["W_out"]["b"])
    probs = pl.pallas_call(
        _probs_body,
        grid=grid,
        in_specs=[hspec, _full_spec((H, 128)), _full_spec((1, 128))],
        out_specs=_node_spec(bn, 128),
        out_shape=jax.ShapeDtypeStruct((BN, 128), _F32),
    )(hv, wo, bo)
    return probs[:, :V].reshape(B, N, V)


# speedup vs baseline: 1.0310x; 1.0310x over previous
import functools

import jax
import jax.numpy as jnp
from jax import lax
from jax.experimental import pallas as pl
from jax.experimental.pallas import tpu as pltpu
from jax.experimental.pallas import tpu_sc as plsc

_F32 = jnp.float32
_BF16 = jnp.bfloat16
_SQRT1_2 = 0.7071067811865476

_BN_BLOCK = 128
_GATHER_CHUNK = 128


def _gelu(x):
    return 0.5 * x * (1.0 + lax.erf(x * _SQRT1_2))


def _ln(x, g, b):
    mu = jnp.mean(x, axis=-1, keepdims=True)
    d = x - mu
    var = jnp.mean(d * d, axis=-1, keepdims=True)
    return g * (d / jnp.sqrt(var + 1e-5)) + b


def _mm(x, w):
    return jnp.dot(x.astype(_BF16), w, preferred_element_type=_F32)


def _sc_gather(table, idx):
    M = idx.shape[0]
    D = table.shape[1]
    n_workers = 32
    per_w = M // n_workers
    n_chunks = per_w // _GATHER_CHUNK
    mesh = plsc.VectorSubcoreMesh(core_axis_name="c", subcore_axis_name="s")

    CH = _GATHER_CHUNK

    @functools.partial(
        pl.kernel,
        mesh=mesh,
        out_type=jax.ShapeDtypeStruct((M, D), table.dtype),
        scratch_types=[
            pltpu.VMEM((per_w,), jnp.int32),
            pltpu.VMEM((CH, D), table.dtype),
            pltpu.VMEM((CH, D), table.dtype),
            pltpu.SemaphoreType.DMA,
            pltpu.SemaphoreType.DMA,
            pltpu.SemaphoreType.DMA,
            pltpu.SemaphoreType.DMA,
        ],
    )
    def k(table_hbm, idx_hbm, out_hbm, idx_v, rows0, rows1, g0, g1, w0, w1):
        wid = lax.axis_index("s") * 2 + lax.axis_index("c")
        base = wid * per_w
        rows = (rows0, rows1)
        gsem = (g0, g1)
        wsem = (w0, w1)
        pltpu.sync_copy(idx_hbm.at[pl.ds(base, per_w)], idx_v)
        pend_g = [None, None]
        pend_w = [None, None]
        for i in range(n_chunks):
            b = i % 2
            if pend_w[b] is not None:
                pend_w[b].wait()
            pend_g[b] = pltpu.async_copy(
                table_hbm.at[idx_v.at[pl.ds(i * CH, CH)]], rows[b], gsem[b])
            if i >= 1:
                pb = 1 - b
                pend_g[pb].wait()
                pend_w[pb] = pltpu.async_copy(
                    rows[pb],
                    out_hbm.at[pl.ds(base + (i - 1) * CH, CH)], wsem[pb])
        bl = (n_chunks - 1) % 2
        pend_g[bl].wait()
        pltpu.sync_copy(rows[bl],
                        out_hbm.at[pl.ds(base + (n_chunks - 1) * CH, CH)])
        if pend_w[1 - bl] is not None:
            pend_w[1 - bl].wait()

    return k(table, idx)


def _node_spec(bn, h):
    return pl.BlockSpec((bn, h), lambda i: (i, 0))


def _full_spec(shape):
    return pl.BlockSpec(shape, lambda i: tuple(0 for _ in shape))


def _embed_body(x_ref, wt_ref, b_ref, of_ref, ob_ref, *, K):
    he = _mm(x_ref[...], wt_ref[...]) + b_ref[...]
    of_ref[...] = he
    ob_ref[...] = he.astype(_BF16)


def _edge_mlp(x1, w2, b2, w3, b3):
    e1 = _gelu(x1.astype(_BF16))
    e2 = _gelu((jnp.dot(e1, w2[...], preferred_element_type=_F32)
                + b2[...]).astype(_BF16))
    return jnp.dot(e2, w3[...], preferred_element_type=_F32) + b3[...]


def _node_update(hv, dh, mv, g1, bb1, win, bin_, wo4, bo, g2, bb2):
    u = _ln(hv + dh, g1[...], bb1[...])
    f1 = _gelu(_mm(u, win[...]) + bin_[...])
    v = _ln(u + _mm(f1, wo4[...]) + bo[...], g2[...], bb2[...])
    return v * mv[...]


def _enc_a0_body(heb_ref, ma_ref, mv_ref,
                 w1b, b1, w2, b2, w3, b3,
                 g1, bb1, win, bin_, wo4, bo, g2, bb2, out_ref, *, K):
    bnk, H = heb_ref.shape
    bn = bnk // K
    t = _mm(heb_ref[...], w1b[...]) + b1[...]
    m = _edge_mlp(t, w2, b2, w3, b3)
    m = m.reshape(bn, K, H) * ma_ref[...][:, :, None]
    dh = jnp.sum(m, axis=1) * (1.0 / 30.0)
    u = _ln(dh, g1[...], bb1[...])
    f1 = _gelu(_mm(u, win[...]) + bin_[...])
    v = _ln(u + _mm(f1, wo4[...]) + bo[...], g2[...], bb2[...])
    out_ref[...] = v * mv_ref[...]


def _enc_a_body(hv_ref, heb_ref, hvg_ref, ma_ref, mv_ref,
                w1a, w1b, w1c, b1, w2, b2, w3, b3,
                g1, bb1, win, bin_, wo4, bo, g2, bb2, out_ref, *, K):
    bn, H = hv_ref.shape
    hv = hv_ref[...]
    t = _mm(heb_ref[...], w1b[...]) + _mm(hvg_ref[...], w1c[...])
    hv1 = _mm(hv, w1a[...])
    x1 = t.reshape(bn, K, H) + hv1[:, None, :] + b1[...][None]
    m = _edge_mlp(x1.reshape(bn * K, H), w2, b2, w3, b3)
    m = m.reshape(bn, K, H) * ma_ref[...][:, :, None]
    dh = jnp.sum(m, axis=1) * (1.0 / 30.0)
    out_ref[...] = _node_update(hv, dh, mv_ref, g1, bb1, win, bin_, wo4, bo,
                                g2, bb2)


def _enc_b_body(hv_ref, hef_ref, hvg_ref,
                w1a, w1b, w1c, b1, w2, b2, w3, b3, g3, bb3,
                of_ref, ob_ref, *, K):
    bn, H = hv_ref.shape
    hef = hef_ref[...]
    t = _mm(hef, w1b[...]) + _mm(hvg_ref[...], w1c[...])
    hv1 = _mm(hv_ref[...], w1a[...])
    x1 = t.reshape(bn, K, H) + hv1[:, None, :] + b1[...][None]
    m = _edge_mlp(x1.reshape(bn * K, H), w2, b2, w3, b3)
    he = _ln(hef + m, g3[...], bb3[...])
    of_ref[...] = he
    ob_ref[...] = he.astype(_BF16)


def _dec_body(hv_ref, heb_ref, hsg_ref, hveg_ref, hvg_ref, m2_ref, mv_ref,
              w1a, w1b, w1c, w1d, b1, w2, b2, w3, b3,
              g1, bb1, win, bin_, wo4, bo, g2, bb2, out_ref, *, K):
    bn, H = hv_ref.shape
    hv = hv_ref[...]
    mvb = mv_ref[...].astype(_BF16)[:, :, None]
    m2b = m2_ref[...].astype(_BF16)[:, :, None]
    m2e = m2b * mvb
    fwe = mvb - m2e
    heb = heb_ref[...].reshape(bn, K, H)
    xb = (heb * mvb).reshape(bn * K, H)
    hsg = hsg_ref[...].astype(_BF16).reshape(bn, K, H)
    hveg = hveg_ref[...].astype(_BF16).reshape(bn, K, H)
    hvg = hvg_ref[...].astype(_BF16).reshape(bn, K, H)
    xc = (hsg * m2e).reshape(bn * K, H)
    xd = (hvg * m2e + hveg * fwe).reshape(bn * K, H)
    t = (jnp.dot(xb, w1b[...], preferred_element_type=_F32)
         + jnp.dot(xc, w1c[...], preferred_element_type=_F32)
         + jnp.dot(xd, w1d[...], preferred_element_type=_F32))
    hv1 = _mm(hv, w1a[...])
    x1 = t.reshape(bn, K, H) + hv1[:, None, :] + b1[...][None]
    m = _edge_mlp(x1.reshape(bn * K, H), w2, b2, w3, b3)
    dh = jnp.sum(m.reshape(bn, K, H), axis=1) * (1.0 / 30.0)
    out_ref[...] = _node_update(hv, dh, mv_ref, g1, bb1, win, bin_, wo4, bo,
                                g2, bb2)


def _probs_body(hv_ref, wo_ref, bo_ref, out_ref):
    logits = _mm(hv_ref[...], wo_ref[...]) + bo_ref[...]
    mx = jnp.max(logits, axis=-1, keepdims=True)
    e = jnp.exp(logits - mx)
    out_ref[...] = e / jnp.sum(e, axis=-1, keepdims=True)


def _wt(p):
    return p["W"].T.astype(_BF16)


def _b2d(p):
    return p["b"][None, :].astype(_F32)


def _n2d(p):
    return p["g"][None, :].astype(_F32), p["b"][None, :].astype(_F32)


def kernel(X, S, mask, E_idx, params):
    B, N, K, EF = X.shape
    H = params["W_e"]["W"].shape[0]
    V = params["W_out"]["W"].shape[0]
    BN = B * N
    M = BN * K
    bn = _BN_BLOCK
    grid = (BN // bn,)
    bnk = bn * K

    offs = (jnp.arange(B, dtype=jnp.int32) * N)[:, None, None]
    flat_idx = (E_idx + offs).reshape(-1)

    mv = mask.reshape(BN, 1)

    rand = jnp.abs(jax.random.normal(jax.random.key(42), (B, N), dtype=_F32))
    decoding_order = jnp.argsort((mask + 0.0001) * rand, axis=-1)
    iota_n = jnp.broadcast_to(jnp.arange(N, dtype=jnp.int32), (B, N))
    pos = (jnp.zeros((B, N), jnp.int32)
           .at[jnp.arange(B)[:, None], decoding_order].set(iota_n)
           .astype(_F32).reshape(BN))

    packed = jnp.zeros((BN, 128), _F32)
    packed = packed.at[:, 0].set(mask.reshape(BN)).at[:, 1].set(pos)
    pg = _sc_gather(packed, flat_idx)
    mask_attend = mv * pg[:, 0].reshape(BN, K)
    m2 = (pos[:, None] > pg[:, 1].reshape(BN, K)).astype(_F32)

    edge_spec = _node_spec(bnk, H)
    hspec = _node_spec(bn, H)
    w_hh = _full_spec((H, H))
    b_h = _full_spec((1, H))

    def enc_a0(heb, p):
        w1t = p["W1"]["W"].T.astype(_BF16)
        g1, bb1 = _n2d(p["norm1"])
        g2, bb2 = _n2d(p["norm2"])
        return pl.pallas_call(
            functools.partial(_enc_a0_body, K=K),
            grid=grid,
            in_specs=[edge_spec, _node_spec(bn, K), _node_spec(bn, 1),
                      w_hh, b_h, w_hh, b_h, w_hh, b_h,
                      b_h, b_h, _full_spec((H, 4 * H)), _full_spec((1, 4 * H)),
                      _full_spec((4 * H, H)), b_h, b_h, b_h],
            out_specs=hspec,
            out_shape=jax.ShapeDtypeStruct((BN, H), _F32),
        )(heb, mask_attend, mv,
          w1t[H:2 * H], _b2d(p["W1"]),
          _wt(p["W2"]), _b2d(p["W2"]), _wt(p["W3"]), _b2d(p["W3"]),
          g1, bb1, _wt(p["dense"]["W_in"]), _b2d(p["dense"]["W_in"]),
          _wt(p["dense"]["W_out"]), _b2d(p["dense"]["W_out"]), g2, bb2)

    def enc_a(hv, heb, hvg, p):
        w1t = p["W1"]["W"].T.astype(_BF16)
        g1, bb1 = _n2d(p["norm1"])
        g2, bb2 = _n2d(p["norm2"])
        return pl.pallas_call(
            functools.partial(_enc_a_body, K=K),
            grid=grid,
            in_specs=[hspec, edge_spec, edge_spec,
                      _node_spec(bn, K), _node_spec(bn, 1),
                      w_hh, w_hh, w_hh, b_h, w_hh, b_h, w_hh, b_h,
                      b_h, b_h, _full_spec((H, 4 * H)), _full_spec((1, 4 * H)),
                      _full_spec((4 * H, H)), b_h, b_h, b_h],
            out_specs=hspec,
            out_shape=jax.ShapeDtypeStruct((BN, H), _F32),
        )(hv, heb, hvg, mask_attend, mv,
          w1t[:H], w1t[H:2 * H], w1t[2 * H:], _b2d(p["W1"]),
          _wt(p["W2"]), _b2d(p["W2"]), _wt(p["W3"]), _b2d(p["W3"]),
          g1, bb1, _wt(p["dense"]["W_in"]), _b2d(p["dense"]["W_in"]),
          _wt(p["dense"]["W_out"]), _b2d(p["dense"]["W_out"]), g2, bb2)

    def enc_b(hv, hef, hvg, p):
        w1t = p["W11"]["W"].T.astype(_BF16)
        g3, bb3 = _n2d(p["norm3"])
        return pl.pallas_call(
            functools.partial(_enc_b_body, K=K),
            grid=grid,
            in_specs=[hspec, edge_spec, edge_spec,
                      w_hh, w_hh, w_hh, b_h, w_hh, b_h, w_hh, b_h, b_h, b_h],
            out_specs=[edge_spec, edge_spec],
            out_shape=[jax.ShapeDtypeStruct((M, H), _F32),
                       jax.ShapeDtypeStruct((M, H), _BF16)],
        )(hv, hef, hvg,
          w1t[:H], w1t[H:2 * H], w1t[2 * H:], _b2d(p["W11"]),
          _wt(p["W12"]), _b2d(p["W12"]), _wt(p["W13"]), _b2d(p["W13"]),
          g3, bb3)

    def dec(hv, heb, hsg, hveg, hvg, p):
        w1t = p["W1"]["W"].T.astype(_BF16)
        g1, bb1 = _n2d(p["norm1"])
        g2, bb2 = _n2d(p["norm2"])
        return pl.pallas_call(
            functools.partial(_dec_body, K=K),
            grid=grid,
            in_specs=[hspec, edge_spec, edge_spec, edge_spec, edge_spec,
                      _node_spec(bn, K), _node_spec(bn, 1),
                      w_hh, w_hh, w_hh, w_hh, b_h, w_hh, b_h, w_hh, b_h,
                      b_h, b_h, _full_spec((H, 4 * H)), _full_spec((1, 4 * H)),
                      _full_spec((4 * H, H)), b_h, b_h, b_h],
            out_specs=hspec,
            out_shape=jax.ShapeDtypeStruct((BN, H), _F32),
        )(hv, heb, hsg, hveg, hvg, m2, mv,
          w1t[:H], w1t[H:2 * H], w1t[2 * H:3 * H], w1t[3 * H:], _b2d(p["W1"]),
          _wt(p["W2"]), _b2d(p["W2"]), _wt(p["W3"]), _b2d(p["W3"]),
          g1, bb1, _wt(p["dense"]["W_in"]), _b2d(p["dense"]["W_in"]),
          _wt(p["dense"]["W_out"]), _b2d(p["dense"]["W_out"]), g2, bb2)

    X2 = X.reshape(M, EF)
    wet = params["W_e"]["W"].T.astype(_BF16)
    hef, heb = pl.pallas_call(
        functools.partial(_embed_body, K=K),
        grid=grid,
        in_specs=[_node_spec(bnk, EF), _full_spec((EF, H)), b_h],
        out_specs=[edge_spec, edge_spec],
        out_shape=[jax.ShapeDtypeStruct((M, H), _F32),
                   jax.ShapeDtypeStruct((M, H), _BF16)],
    )(X2, wet, _b2d(params["W_e"]))

    hv = hvg = None
    for i, p in enumerate(params["enc"]):
        hv = enc_a0(heb, p) if i == 0 else enc_a(hv, heb, hvg, p)
        hvg = _sc_gather(hv, flat_idx)
        hef, heb = enc_b(hv, hef, hvg, p)

    hveg = hvg
    hs = jnp.take(params["W_s"], S.reshape(-1), axis=0)
    hsg = _sc_gather(hs, flat_idx)

    for i, p in enumerate(params["dec"]):
        hvg_d = hveg if i == 0 else _sc_gather(hv, flat_idx)
        hv = dec(hv, heb, hsg, hveg, hvg_d, p)

    wo = jnp.zeros((H, 128), _F32).at[:, :V].set(
        params["W_out"]["W"].T).astype(_BF16)
    bo = jnp.full((1, 128), -1e9, _F32).at[0, :V].set(params["W_out"]["b"])
    probs = pl.pallas_call(
        _probs_body,
        grid=grid,
        in_specs=[hspec, _full_spec((H, 128)), _full_spec((1, 128))],
        out_specs=_node_spec(bn, 128),
        out_shape=jax.ShapeDtypeStruct((BN, 128), _F32),
    )(hv, wo, bo)
    return probs[:, :V].reshape(B, N, V)
